# baseline (speedup 1.0000x reference)
import functools

import jax
import jax.numpy as jnp
from jax import lax
from jax.experimental import pallas as pl
from jax.experimental.pallas import tpu as pltpu
from jax.experimental.pallas import tpu_sc as plsc

BATCH = 16
LENGTH = 512
N_CHANNELS = 6
N_REL = 131072
TOTAL = BATCH * LENGTH * LENGTH * N_CHANNELS

NUM_CORES = 2
NUM_SUBCORES = 16
HALF = TOTAL // NUM_CORES
NWIN = 2
WWORDS = HALF // NWIN
WTILE = WWORDS // NUM_SUBCORES
ZCHUNK = 32768
NZD_W = WTILE // ZCHUNK

ICOLS = 128
REL_PER_TILE = N_REL // NUM_SUBCORES
IROWS = REL_PER_TILE // ICOLS
VSTEPS = REL_PER_TILE // 16
IGNORED = -1
DMA_GROUP = 8

_mesh = plsc.VectorSubcoreMesh(
    core_axis_name="c", subcore_axis_name="s",
    num_cores=NUM_CORES, num_subcores=NUM_SUBCORES,
)


@functools.partial(
    pl.kernel,
    mesh=_mesh,
    out_type=jax.ShapeDtypeStruct((TOTAL,), jnp.float32),
    scratch_types=[
        pltpu.VMEM((ZCHUNK,), jnp.float32),
        pltpu.VMEM((REL_PER_TILE,), jnp.int32),
        pltpu.VMEM((REL_PER_TILE,), jnp.int32),
        pltpu.VMEM((REL_PER_TILE,), jnp.int32),
        pltpu.VMEM((REL_PER_TILE,), jnp.int32),
        pltpu.VMEM((REL_PER_TILE,), jnp.float32),
        pltpu.VMEM((REL_PER_TILE,), jnp.int32),
        pltpu.VMEM((REL_PER_TILE,), jnp.int32),
        pltpu.SemaphoreType.DMA,
        pltpu.SemaphoreType.DMA,
        pltpu.SemaphoreType.DMA,
    ],
)
def _sc_dep(rb_hbm, rs_hbm, rd_hbm, rt_hbm, rv_hbm, dep_hbm,
            zero_v, b_v, s_v, d_v, t_v, v_v, idx0_v, idx1_v,
            zsem0, zsem1, sem):
    cid = lax.axis_index("c")
    sid = lax.axis_index("s")

    def fill_zero(i, carry):
        zero_v[pl.ds(i * 16, 16)] = jnp.zeros((16,), jnp.float32)
        return carry

    lax.fori_loop(0, ZCHUNK // 16, fill_zero, 0)

    lo = cid * HALF
    zero_copies = [[], []]
    for w, zsem in ((0, zsem0), (1, zsem1)):
        wbase = lo + w * WWORDS + sid * WTILE
        for k in range(NZD_W):
            cp = pltpu.make_async_copy(
                zero_v, dep_hbm.at[pl.ds(wbase + k * ZCHUNK, ZCHUNK)], zsem
            )
            cp.start()
            zero_copies[w].append(cp)

    rel_base = sid * REL_PER_TILE
    in_copies = [
        pltpu.async_copy(hbm.at[pl.ds(rel_base, REL_PER_TILE)], vmem, sem)
        for hbm, vmem in [(rb_hbm, b_v), (rs_hbm, s_v), (rd_hbm, d_v),
                          (rt_hbm, t_v), (rv_hbm, v_v)]
    ]
    for cp in in_copies:
        cp.wait()

    mid = lo + WWORDS
    hi = lo + HALF
    ignored16 = jnp.full((16,), IGNORED, jnp.int32)

    def compute(m, carry):
        p = m * 16
        b = b_v[pl.ds(p, 16)]
        s = s_v[pl.ds(p, 16)]
        d = d_v[pl.ds(p, 16)]
        t = t_v[pl.ds(p, 16)]
        off = (
            (b * N_CHANNELS + t) * (LENGTH * LENGTH)
            + (((s >> 3) << 2) + (d >> 7)) * 1024
            + ((s & 7) << 7)
            + (d & 127)
        )
        in0 = (off >= lo) & (off < mid)
        in1 = (off >= mid) & (off < hi)
        idx0_v[pl.ds(p, 16)] = jnp.where(in0, off, ignored16)
        idx1_v[pl.ds(p, 16)] = jnp.where(in1, off, ignored16)
        return carry

    lax.fori_loop(0, VSTEPS, compute, 0)

    for w, idx_v in ((0, idx0_v), (1, idx1_v)):
        for cp in zero_copies[w]:
            cp.wait()
        plsc.subcore_barrier()

        def fire(g, carry, idx_v=idx_v):
            row0 = g * DMA_GROUP
            for j in range(DMA_GROUP):
                pltpu.make_async_copy(
                    v_v.at[pl.ds((row0 + j) * ICOLS, ICOLS)],
                    dep_hbm.at[plsc.Indices(
                        idx_v.at[pl.ds((row0 + j) * ICOLS, ICOLS)],
                        ignored_value=IGNORED)],
                    sem,
                ).start()
            return carry

        lax.fori_loop(0, IROWS // DMA_GROUP, fire, 0)

    def drain(g, carry):
        for _ in range(DMA_GROUP):
            pltpu.make_async_copy(
                v_v.at[pl.ds(0, ICOLS)],
                dep_hbm.at[plsc.Indices(idx0_v.at[pl.ds(0, ICOLS)],
                                        ignored_value=IGNORED)],
                sem,
            ).wait()
        return carry

    lax.fori_loop(0, NWIN * IROWS // DMA_GROUP, drain, 0)


def kernel(rel_b, rel_src, rel_dst, rel_type, rel_val):
    dep = _sc_dep(rel_b, rel_src, rel_dst, rel_type, rel_val)
    x = dep.reshape(BATCH, N_CHANNELS, LENGTH // 8, LENGTH // 128, 8, 128)
    x = x.transpose(0, 2, 4, 3, 5, 1)
    return x.reshape(BATCH, LENGTH, LENGTH, N_CHANNELS)


# speedup vs baseline: 1.0074x; 1.0074x over previous
import functools

import jax
import jax.numpy as jnp
from jax import lax
from jax.experimental import pallas as pl
from jax.experimental.pallas import tpu as pltpu
from jax.experimental.pallas import tpu_sc as plsc

BATCH = 16
LENGTH = 512
N_CHANNELS = 6
N_REL = 131072
TOTAL = BATCH * LENGTH * LENGTH * N_CHANNELS

NUM_CORES = 2
NUM_SUBCORES = 16
HALF = TOTAL // NUM_CORES
SHARD = HALF // NUM_SUBCORES
ZCHUNK = 32768
NZDMA = SHARD // ZCHUNK

ICOLS = 128
REL_PER_TILE = N_REL // NUM_SUBCORES
IROWS = REL_PER_TILE // ICOLS
VSTEPS = REL_PER_TILE // 16
IGNORED = -1
DMA_GROUP = 8

_mesh = plsc.VectorSubcoreMesh(
    core_axis_name="c", subcore_axis_name="s",
    num_cores=NUM_CORES, num_subcores=NUM_SUBCORES,
)


@functools.partial(
    pl.kernel,
    mesh=_mesh,
    out_type=jax.ShapeDtypeStruct((TOTAL,), jnp.float32),
    scratch_types=[
        pltpu.VMEM((ZCHUNK,), jnp.float32),
        pltpu.VMEM((REL_PER_TILE,), jnp.int32),
        pltpu.VMEM((REL_PER_TILE,), jnp.int32),
        pltpu.VMEM((REL_PER_TILE,), jnp.int32),
        pltpu.VMEM((REL_PER_TILE,), jnp.int32),
        pltpu.VMEM((REL_PER_TILE,), jnp.float32),
        pltpu.VMEM((IROWS, ICOLS), jnp.int32),
        pltpu.SemaphoreType.DMA,
        pltpu.SemaphoreType.DMA,
    ],
)
def _sc_dep(rb_hbm, rs_hbm, rd_hbm, rt_hbm, rv_hbm, dep_hbm,
            zero_v, b_v, s_v, d_v, t_v, v_v, idx_v, zsem, sem):
    cid = lax.axis_index("c")
    sid = lax.axis_index("s")

    def fill_zero(i, carry):
        zero_v[pl.ds(i * 16, 16)] = jnp.zeros((16,), jnp.float32)
        return carry

    lax.fori_loop(0, ZCHUNK // 16, fill_zero, 0)

    shard_base = cid * HALF + sid * SHARD
    zero_copies = [
        pltpu.make_async_copy(
            zero_v, dep_hbm.at[pl.ds(shard_base + k * ZCHUNK, ZCHUNK)], zsem
        )
        for k in range(NZDMA)
    ]
    for cp in zero_copies:
        cp.start()

    rel_base = sid * REL_PER_TILE
    in_copies = [
        pltpu.async_copy(hbm.at[pl.ds(rel_base, REL_PER_TILE)], vmem, sem)
        for hbm, vmem in [(rb_hbm, b_v), (rs_hbm, s_v), (rd_hbm, d_v),
                          (rt_hbm, t_v), (rv_hbm, v_v)]
    ]
    for cp in in_copies:
        cp.wait()

    lo = cid * HALF
    hi = lo + HALF

    def compute(m, carry):
        p = m * 16
        b = b_v[pl.ds(p, 16)]
        s = s_v[pl.ds(p, 16)]
        d = d_v[pl.ds(p, 16)]
        t = t_v[pl.ds(p, 16)]
        off = (
            (b * N_CHANNELS + t) * (LENGTH * LENGTH)
            + (((s >> 3) << 2) + (d >> 7)) * 1024
            + ((s & 7) << 7)
            + (d & 127)
        )
        owned = (off >= lo) & (off < hi)
        r = m // 8
        c16 = (m % 8) * 16
        idx_v[r, pl.ds(c16, 16)] = jnp.where(
            owned, off, jnp.full((16,), IGNORED, jnp.int32)
        )
        return carry

    lax.fori_loop(0, VSTEPS, compute, 0)

    for cp in zero_copies:
        cp.wait()
    plsc.subcore_barrier()

    def group(g, carry):
        row0 = g * DMA_GROUP
        copies = [
            pltpu.async_copy(
                v_v.at[pl.ds((row0 + j) * ICOLS, ICOLS)],
                dep_hbm.at[plsc.Indices(idx_v.at[row0 + j],
                                        ignored_value=IGNORED)],
                sem,
            )
            for j in range(DMA_GROUP)
        ]
        for cp in copies:
            cp.wait()
        return carry

    lax.fori_loop(0, IROWS // DMA_GROUP, group, 0)


def kernel(rel_b, rel_src, rel_dst, rel_type, rel_val):
    dep = _sc_dep(rel_b, rel_src, rel_dst, rel_type, rel_val)
    x = dep.reshape(BATCH, N_CHANNELS, LENGTH // 8, LENGTH // 128, 8, 128)
    x = x.transpose(0, 2, 4, 3, 5, 1)
    return x.reshape(BATCH, LENGTH, LENGTH, N_CHANNELS)
